# baseline (speedup 1.0000x reference)
import functools

import jax
import jax.numpy as jnp
from jax import lax
from jax.experimental import pallas as pl
from jax.experimental.pallas import tpu as pltpu
from jax.experimental.pallas import tpu_sc as plsc

NRP = 24
MAXZ = 84
CHUNK = 2048


@functools.lru_cache(maxsize=None)
def _make_gather(B: int):
    info = plsc.get_sparse_core_info()
    nw = info.num_cores * info.num_subcores
    b_per_w = B // nw
    n_chunks = b_per_w // CHUNK
    assert b_per_w % CHUNK == 0 and n_chunks % 2 == 0 and n_chunks >= 4
    c16 = CHUNK // 16

    mesh = plsc.VectorSubcoreMesh(core_axis_name="c", subcore_axis_name="s")

    @functools.partial(
        pl.kernel,
        mesh=mesh,
        compiler_params=pltpu.CompilerParams(needs_layout_passes=False),
        out_type=jax.ShapeDtypeStruct((NRP, B), jnp.float32),
        scratch_types=[
            pltpu.VMEM((MAXZ * NRP,), jnp.float32),
            pltpu.VMEM((CHUNK,), jnp.int32),
            pltpu.VMEM((CHUNK,), jnp.int32),
            pltpu.VMEM((NRP, CHUNK), jnp.float32),
            pltpu.VMEM((NRP, CHUNK), jnp.float32),
            pltpu.SemaphoreType.DMA,
            pltpu.SemaphoreType.DMA,
            pltpu.SemaphoreType.DMA,
            pltpu.SemaphoreType.DMA,
        ],
    )
    def gather_kernel(z_hbm, p_hbm, out_hbm, p_v, z0, z1, r0, r1,
                      sz0, sz1, so0, so1):
        wid = lax.axis_index("s") * info.num_cores + lax.axis_index("c")
        base = wid * b_per_w
        zs, rs, szs, sos = (z0, z1), (r0, r1), (sz0, sz1), (so0, so1)

        pltpu.sync_copy(p_hbm, p_v)

        def z_start(g, b):
            pltpu.make_async_copy(
                z_hbm.at[pl.ds(base + g * CHUNK, CHUNK)], zs[b], szs[b]
            ).start()

        def z_wait(b):
            pltpu.make_async_copy(
                z_hbm.at[pl.ds(0, CHUNK)], zs[b], szs[b]).wait()

        def o_start(g, b):
            pltpu.make_async_copy(
                rs[b], out_hbm.at[:, pl.ds(base + g * CHUNK, CHUNK)],
                sos[b]).start()

        def o_wait(b):
            pltpu.make_async_copy(
                rs[b], out_hbm.at[:, pl.ds(0, CHUNK)], sos[b]).wait()

        def compute(zr, rr):
            @plsc.parallel_loop(0, c16, 1, unroll=16)
            def _(a):
                a0 = a * 16
                zv24 = zr[pl.ds(a0, 16)] * NRP
                vals = [plsc.load_gather(p_v, [zv24 + j])
                        for j in range(NRP)]
                for j in range(NRP):
                    rr[j, pl.ds(a0, 16)] = vals[j]

        z_start(0, 0)
        z_start(1, 1)
        z_wait(0)
        compute(z0, r0)
        o_start(0, 0)
        z_start(2, 0)
        z_wait(1)
        compute(z1, r1)
        o_start(1, 1)
        z_start(3, 1)

        def pair(i, carry):
            for b in (0, 1):
                g = 2 * i + b
                z_wait(b)
                o_wait(b)
                compute(zs[b], rs[b])
                o_start(g, b)

                @pl.when(g + 2 < n_chunks)
                def _():
                    z_start(g + 2, b)
            return carry

        lax.fori_loop(1, n_chunks // 2, pair, 0)
        o_wait(0)
        o_wait(1)

    return gather_kernel


def kernel(Z, p, alpha, chi):
    B = Z.shape[0]
    zi = Z.astype(jnp.int32)
    gathered_t = _make_gather(B)(zi, p.reshape(-1))
    return (gathered_t.T, alpha, chi)


# speedup vs baseline: 1.0133x; 1.0133x over previous
import functools

import jax
import jax.numpy as jnp
from jax import lax
from jax.experimental import pallas as pl
from jax.experimental.pallas import tpu as pltpu
from jax.experimental.pallas import tpu_sc as plsc

NRP = 24
MAXZ = 84
CHUNK = 2048


@functools.lru_cache(maxsize=None)
def _make_gather(B: int):
    info = plsc.get_sparse_core_info()
    nw = info.num_cores * info.num_subcores
    b_per_w = B // nw
    n_chunks = b_per_w // CHUNK
    assert b_per_w % CHUNK == 0 and n_chunks % 2 == 0 and n_chunks >= 4
    c16 = CHUNK // 16

    mesh = plsc.VectorSubcoreMesh(core_axis_name="c", subcore_axis_name="s")

    @functools.partial(
        pl.kernel,
        mesh=mesh,
        compiler_params=pltpu.CompilerParams(needs_layout_passes=False),
        out_type=jax.ShapeDtypeStruct((NRP, B), jnp.float32),
        scratch_types=[
            pltpu.VMEM((MAXZ * NRP,), jnp.float32),
            pltpu.VMEM((CHUNK,), jnp.int32),
            pltpu.VMEM((CHUNK,), jnp.int32),
            pltpu.VMEM((NRP, CHUNK), jnp.float32),
            pltpu.VMEM((NRP, CHUNK), jnp.float32),
            pltpu.SemaphoreType.DMA,
            pltpu.SemaphoreType.DMA,
            pltpu.SemaphoreType.DMA,
            pltpu.SemaphoreType.DMA,
        ],
    )
    def gather_kernel(z_hbm, p_hbm, out_hbm, p_v, z0, z1, r0, r1,
                      sz0, sz1, so0, so1):
        wid = lax.axis_index("s") * info.num_cores + lax.axis_index("c")
        base = wid * b_per_w
        zs, rs, szs, sos = (z0, z1), (r0, r1), (sz0, sz1), (so0, so1)

        pltpu.sync_copy(p_hbm, p_v)

        def z_start(g, b):
            pltpu.make_async_copy(
                z_hbm.at[pl.ds(base + g * CHUNK, CHUNK)], zs[b], szs[b]
            ).start()

        def z_wait(b):
            pltpu.make_async_copy(
                z_hbm.at[pl.ds(0, CHUNK)], zs[b], szs[b]).wait()

        def o_start(g, b):
            pltpu.make_async_copy(
                rs[b], out_hbm.at[:, pl.ds(base + g * CHUNK, CHUNK)],
                sos[b]).start()

        def o_wait(b):
            pltpu.make_async_copy(
                rs[b], out_hbm.at[:, pl.ds(0, CHUNK)], sos[b]).wait()

        def compute(zr, rr):
            @plsc.parallel_loop(0, c16, 1, unroll=8)
            def _(a):
                a0 = a * 16
                zv24 = zr[pl.ds(a0, 16)] * NRP
                vals = [plsc.load_gather(p_v, [zv24 + j])
                        for j in range(NRP)]
                for j in range(NRP):
                    rr[j, pl.ds(a0, 16)] = vals[j]

        z_start(0, 0)
        z_start(1, 1)
        z_wait(0)
        compute(z0, r0)
        o_start(0, 0)
        z_start(2, 0)
        z_wait(1)
        compute(z1, r1)
        o_start(1, 1)
        z_start(3, 1)

        def pair(i, carry):
            for b in (0, 1):
                g = 2 * i + b
                z_wait(b)
                o_wait(b)
                compute(zs[b], rs[b])
                o_start(g, b)

                @pl.when(g + 2 < n_chunks)
                def _():
                    z_start(g + 2, b)
            return carry

        lax.fori_loop(1, n_chunks // 2, pair, 0)
        o_wait(0)
        o_wait(1)

    return gather_kernel


def kernel(Z, p, alpha, chi):
    B = Z.shape[0]
    zi = Z.astype(jnp.int32)
    gathered_t = _make_gather(B)(zi, p.reshape(-1))
    return (gathered_t.T, alpha, chi)
